# baseline (speedup 1.0000x reference)
import dataclasses

import jax
import jax.numpy as jnp
from jax import lax
from jax.experimental import pallas as pl
from jax.experimental.pallas import tpu as pltpu
from jax.experimental.pallas import tpu_sc as plsc

N_NODES = 10000
N_EDGES = 160000
DIM = 256
LANES = 16
N_WORKERS = 32
B_W = N_EDGES // N_WORKERS
CHUNK = 96
NF = B_W // CHUNK
TAIL = B_W - NF * CHUNK
TAIL_GROUPS = -(-TAIL // LANES)


def _dot_group(rows_s, rows_d, i0, lane):
    def sub(s, outv):
        for j in range(4):
            e = s * 4 + j
            i = i0 + e
            ps = [rows_s[i, pl.ds(cd * LANES, LANES)]
                  * rows_d[i, pl.ds(cd * LANES, LANES)]
                  for cd in range(DIM // LANES)]
            while len(ps) > 1:
                ps = [ps[a] + ps[a + 1] for a in range(0, len(ps), 2)]
            tot = jnp.sum(ps[0])
            outv = jnp.where(lane == e, tot, outv)
        return outv

    outv = lax.fori_loop(0, 4, sub, jnp.zeros((LANES,), jnp.float32))
    return 1.0 / (1.0 + jnp.exp(-outv))


def _sc_body(z_hbm, srci_hbm, dsti_hbm, out_hbm,
             idx_s, idx_d, rows_s0, rows_d0, rows_s1, rows_d1, out0, out1,
             sem_gs0, sem_gd0, sem_gs1, sem_gd1, sem_o0, sem_o1):
    rows_s = (rows_s0, rows_s1)
    rows_d = (rows_d0, rows_d1)
    out_v = (out0, out1)
    sem_gs = (sem_gs0, sem_gs1)
    sem_gd = (sem_gd0, sem_gd1)
    sem_o = (sem_o0, sem_o1)

    cid = lax.axis_index("c")
    sid = lax.axis_index("s")
    wid = sid * 2 + cid
    base_e = wid * B_W
    lane = lax.iota(jnp.int32, LANES)

    pltpu.sync_copy(srci_hbm.at[pl.ds(base_e, B_W)], idx_s)
    pltpu.sync_copy(dsti_hbm.at[pl.ds(base_e, B_W)], idx_d)

    def start_gather(k, b):
        off = pl.multiple_of(k * CHUNK, 8)
        pltpu.async_copy(z_hbm.at[idx_s.at[pl.ds(off, CHUNK)]],
                         rows_s[b], sem_gs[b])
        pltpu.async_copy(z_hbm.at[idx_d.at[pl.ds(off, CHUNK)]],
                         rows_d[b], sem_gd[b])

    def wait_gather(b):
        pltpu.make_async_copy(z_hbm.at[pl.ds(0, CHUNK)], rows_s[b],
                              sem_gs[b]).wait()
        pltpu.make_async_copy(z_hbm.at[pl.ds(0, CHUNK)], rows_d[b],
                              sem_gd[b]).wait()

    def wait_store(b):
        pltpu.make_async_copy(out_v[b], out_hbm.at[pl.ds(0, CHUNK)],
                              sem_o[b]).wait()

    start_gather(0, 0)
    start_gather(1, 1)

    @pl.loop(0, NF, step=2)
    def _pair(k):
        for b in range(2):
            kk = k + b
            wait_gather(b)
            @pl.when(kk >= 2)
            def _():
                wait_store(b)

            @pl.loop(0, CHUNK, step=LANES)
            def _group(i0):
                out_v[b][pl.ds(i0, LANES)] = _dot_group(
                    rows_s[b], rows_d[b], i0, lane)

            off = pl.multiple_of(base_e + kk * CHUNK, 8)
            pltpu.async_copy(out_v[b], out_hbm.at[pl.ds(off, CHUNK)],
                             sem_o[b])

            @pl.when(kk + 2 < NF)
            def _():
                start_gather(kk + 2, b)

    wait_store(0)
    wait_store(1)

    t_off = NF * CHUNK
    g_s = pltpu.async_copy(z_hbm.at[idx_s.at[pl.ds(t_off, TAIL)]],
                           rows_s[0].at[pl.ds(0, TAIL)], sem_gs[0])
    g_d = pltpu.async_copy(z_hbm.at[idx_d.at[pl.ds(t_off, TAIL)]],
                           rows_d[0].at[pl.ds(0, TAIL)], sem_gd[0])
    g_s.wait()
    g_d.wait()
    for g in range(TAIL_GROUPS):
        out_v[0][pl.ds(g * LANES, LANES)] = _dot_group(
            rows_s[0], rows_d[0], g * LANES, lane)
    pltpu.sync_copy(out_v[0].at[pl.ds(0, TAIL)],
                    out_hbm.at[pl.ds(base_e + t_off, TAIL)])


def _make_sc_kernel():
    mesh = plsc.VectorSubcoreMesh(core_axis_name="c", subcore_axis_name="s")
    cp = pltpu.CompilerParams()
    if "needs_layout_passes" in pltpu.CompilerParams.__dataclass_fields__:
        cp = dataclasses.replace(cp, needs_layout_passes=False)
    return pl.kernel(
        _sc_body,
        out_type=jax.ShapeDtypeStruct((N_EDGES,), jnp.float32),
        mesh=mesh,
        scratch_types=[
            pltpu.VMEM((B_W,), jnp.int32),
            pltpu.VMEM((B_W,), jnp.int32),
            pltpu.VMEM((CHUNK, DIM), jnp.float32),
            pltpu.VMEM((CHUNK, DIM), jnp.float32),
            pltpu.VMEM((CHUNK, DIM), jnp.float32),
            pltpu.VMEM((CHUNK, DIM), jnp.float32),
            pltpu.VMEM((CHUNK,), jnp.float32),
            pltpu.VMEM((CHUNK,), jnp.float32),
            pltpu.SemaphoreType.DMA,
            pltpu.SemaphoreType.DMA,
            pltpu.SemaphoreType.DMA,
            pltpu.SemaphoreType.DMA,
            pltpu.SemaphoreType.DMA,
            pltpu.SemaphoreType.DMA,
        ],
        compiler_params=cp,
    )


_sc_kernel = _make_sc_kernel()


def kernel(z, edge_index):
    ei = edge_index.astype(jnp.int32)
    return _sc_kernel(z, ei[0], ei[1])


# speedup vs baseline: 1.1025x; 1.1025x over previous
import dataclasses

import jax
import jax.numpy as jnp
from jax import lax
from jax.experimental import pallas as pl
from jax.experimental.pallas import tpu as pltpu
from jax.experimental.pallas import tpu_sc as plsc

N_NODES = 10000
N_EDGES = 160000
DIM = 256
LANES = 16
N_WORKERS = 32
B_W = N_EDGES // N_WORKERS
CHUNK = 96
NF = B_W // CHUNK
TAIL = B_W - NF * CHUNK
TAIL_GROUPS = -(-TAIL // LANES)


def _dot_group(rows_s, rows_d, i0, lane):
    outv = jnp.zeros((LANES,), jnp.float32)
    for e in range(LANES):
        i = i0 + e
        acc = rows_s[i, pl.ds(0, LANES)] * rows_d[i, pl.ds(0, LANES)]
        for cd in range(1, DIM // LANES):
            acc = acc + (rows_s[i, pl.ds(cd * LANES, LANES)]
                         * rows_d[i, pl.ds(cd * LANES, LANES)])
        tot = jnp.sum(acc)
        outv = jnp.where(lane == e, tot, outv)
    return 1.0 / (1.0 + jnp.exp(-outv))


def _sc_body(z_hbm, srci_hbm, dsti_hbm, out_hbm,
             idx_s, idx_d, rows_s0, rows_d0, rows_s1, rows_d1, out0, out1,
             sem_gs0, sem_gd0, sem_gs1, sem_gd1, sem_o0, sem_o1):
    rows_s = (rows_s0, rows_s1)
    rows_d = (rows_d0, rows_d1)
    out_v = (out0, out1)
    sem_gs = (sem_gs0, sem_gs1)
    sem_gd = (sem_gd0, sem_gd1)
    sem_o = (sem_o0, sem_o1)

    cid = lax.axis_index("c")
    sid = lax.axis_index("s")
    wid = sid * 2 + cid
    base_e = wid * B_W
    lane = lax.iota(jnp.int32, LANES)

    pltpu.sync_copy(srci_hbm.at[pl.ds(base_e, B_W)], idx_s)
    pltpu.sync_copy(dsti_hbm.at[pl.ds(base_e, B_W)], idx_d)

    def start_gather(k, b):
        off = pl.multiple_of(k * CHUNK, 8)
        pltpu.async_copy(z_hbm.at[idx_s.at[pl.ds(off, CHUNK)]],
                         rows_s[b], sem_gs[b])
        pltpu.async_copy(z_hbm.at[idx_d.at[pl.ds(off, CHUNK)]],
                         rows_d[b], sem_gd[b])

    def wait_gather(b):
        pltpu.make_async_copy(z_hbm.at[pl.ds(0, CHUNK)], rows_s[b],
                              sem_gs[b]).wait()
        pltpu.make_async_copy(z_hbm.at[pl.ds(0, CHUNK)], rows_d[b],
                              sem_gd[b]).wait()

    def wait_store(b):
        pltpu.make_async_copy(out_v[b], out_hbm.at[pl.ds(0, CHUNK)],
                              sem_o[b]).wait()

    start_gather(0, 0)
    start_gather(1, 1)

    @pl.loop(0, NF, step=2)
    def _pair(k):
        for b in range(2):
            kk = k + b
            wait_gather(b)
            @pl.when(kk >= 2)
            def _():
                wait_store(b)

            @pl.loop(0, CHUNK, step=LANES)
            def _group(i0):
                out_v[b][pl.ds(i0, LANES)] = _dot_group(
                    rows_s[b], rows_d[b], i0, lane)

            off = pl.multiple_of(base_e + kk * CHUNK, 8)
            pltpu.async_copy(out_v[b], out_hbm.at[pl.ds(off, CHUNK)],
                             sem_o[b])

            @pl.when(kk + 2 < NF)
            def _():
                start_gather(kk + 2, b)

    wait_store(0)
    wait_store(1)

    t_off = NF * CHUNK
    g_s = pltpu.async_copy(z_hbm.at[idx_s.at[pl.ds(t_off, TAIL)]],
                           rows_s[0].at[pl.ds(0, TAIL)], sem_gs[0])
    g_d = pltpu.async_copy(z_hbm.at[idx_d.at[pl.ds(t_off, TAIL)]],
                           rows_d[0].at[pl.ds(0, TAIL)], sem_gd[0])
    g_s.wait()
    g_d.wait()
    for g in range(TAIL_GROUPS):
        out_v[0][pl.ds(g * LANES, LANES)] = _dot_group(
            rows_s[0], rows_d[0], g * LANES, lane)
    pltpu.sync_copy(out_v[0].at[pl.ds(0, TAIL)],
                    out_hbm.at[pl.ds(base_e + t_off, TAIL)])


def _make_sc_kernel():
    mesh = plsc.VectorSubcoreMesh(core_axis_name="c", subcore_axis_name="s")
    cp = pltpu.CompilerParams()
    if "needs_layout_passes" in pltpu.CompilerParams.__dataclass_fields__:
        cp = dataclasses.replace(cp, needs_layout_passes=False)
    return pl.kernel(
        _sc_body,
        out_type=jax.ShapeDtypeStruct((N_EDGES,), jnp.float32),
        mesh=mesh,
        scratch_types=[
            pltpu.VMEM((B_W,), jnp.int32),
            pltpu.VMEM((B_W,), jnp.int32),
            pltpu.VMEM((CHUNK, DIM), jnp.float32),
            pltpu.VMEM((CHUNK, DIM), jnp.float32),
            pltpu.VMEM((CHUNK, DIM), jnp.float32),
            pltpu.VMEM((CHUNK, DIM), jnp.float32),
            pltpu.VMEM((CHUNK,), jnp.float32),
            pltpu.VMEM((CHUNK,), jnp.float32),
            pltpu.SemaphoreType.DMA,
            pltpu.SemaphoreType.DMA,
            pltpu.SemaphoreType.DMA,
            pltpu.SemaphoreType.DMA,
            pltpu.SemaphoreType.DMA,
            pltpu.SemaphoreType.DMA,
        ],
        compiler_params=cp,
    )


_sc_kernel = _make_sc_kernel()


def kernel(z, edge_index):
    ei = edge_index.astype(jnp.int32)
    return _sc_kernel(z, ei[0], ei[1])
